# baseline (speedup 1.0000x reference)
import functools

import jax
import jax.numpy as jnp
from jax import lax
from jax.experimental import pallas as pl
from jax.experimental.pallas import tpu as pltpu
from jax.experimental.pallas import tpu_sc as plsc

N = 10000
E = 160000
G = 256
NP = 10240
GP = 512
NSC = 2
NT = 16
NW = NSC * NT

ACC_A = 55296
CH_A = ACC_A // NT
KA = 208
EA = NW * KA * 128

KCR0, KCR1 = 208, 112
KCT0, KCT1 = 64, 32
SR_R = 16 * (KCR0 + KCR1) + KCR0
SR_T = 16 * (KCT0 + KCT1) + KCT0
ROWS_R = 4 * NP
ZR_R = ROWS_R // NT
ZR_T = NP // NT

KE = 84
ACC_E = 64 * NP
CH_E = ACC_E // NT


@functools.cache
def _sc_kernels():
    mesh = plsc.VectorSubcoreMesh(core_axis_name="c", subcore_axis_name="s",
                                  num_cores=NSC, num_subcores=NT)

    @functools.partial(
        pl.kernel,
        out_type=jax.ShapeDtypeStruct((NSC, ACC_A), jnp.float32),
        mesh=mesh,
        compiler_params=pltpu.CompilerParams(use_tc_tiling_on_sc=False),
        scratch_types=[
            pltpu.VMEM((KA, 128), jnp.int32),
            pltpu.VMEM((128,), jnp.float32),
            pltpu.VMEM_SHARED((ACC_A,), jnp.float32),
            pltpu.SemaphoreType.DMA,
        ],
    )
    def sc_degree(idx_hbm, zeros_hbm, ones_hbm, out_hbm, idx_v, ones_v,
                  acc_sh, sem):
        if True:
            cid = lax.axis_index("c")
            sid = lax.axis_index("s")
            wid = sid * NSC + cid
            pltpu.sync_copy(zeros_hbm, acc_sh.at[pl.ds(sid * CH_A, CH_A)])
            pltpu.sync_copy(ones_hbm, ones_v)
            pltpu.sync_copy(idx_hbm.at[wid], idx_v)
            plsc.subcore_barrier()

            def body(j, c):
                pltpu.async_copy(ones_v, acc_sh.at[idx_v.at[j]], sem,
                                 add=True)
                return c

            lax.fori_loop(0, KA, body, 0)

            def dbody(j, c):
                pltpu.make_async_copy(ones_v, acc_sh.at[idx_v.at[j]],
                                      sem).wait()
                return c

            lax.fori_loop(0, KA, dbody, 0)
            plsc.subcore_barrier()
            pltpu.sync_copy(acc_sh.at[pl.ds(sid * CH_A, CH_A)],
                            out_hbm.at[cid].at[pl.ds(sid * CH_A, CH_A)])

    @functools.partial(
        pl.kernel,
        out_type=(
            jax.ShapeDtypeStruct((4 * NP, 16), jnp.float32),
            jax.ShapeDtypeStruct((16, 4 * NP), jnp.float32),
            jax.ShapeDtypeStruct((NP, 48), jnp.float32),
            jax.ShapeDtypeStruct((48, NP), jnp.float32),
        ),
        mesh=mesh,
        compiler_params=pltpu.CompilerParams(use_tc_tiling_on_sc=False,
                                             needs_layout_passes=False),
        scratch_types=[
            pltpu.VMEM((1280, 16), jnp.float32),
            pltpu.VMEM((1280,), jnp.float32),
            pltpu.VMEM((16, 128), jnp.float32),
            pltpu.VMEM((640, 48), jnp.float32),
            pltpu.VMEM((640,), jnp.float32),
            pltpu.VMEM((48, 128), jnp.float32),
        ],
    )
    def sc_build(xr_hbm, xt_hbm, dis_hbm, y1r_hbm, y1rt_hbm, y1t_hbm,
                 y1tt_hbm, xb, db, tb, xtb, dtb, ttb):
        cid = lax.axis_index("c")
        sid = lax.axis_index("s")
        wid = sid * NSC + cid
        iota16 = lax.broadcasted_iota(jnp.int32, (16,), 0)
        r0 = wid * 1280
        pltpu.sync_copy(xr_hbm.at[pl.ds(r0, 1280)], xb)
        pltpu.sync_copy(dis_hbm.at[pl.ds(r0, 1280)], db)

        def rbody(r, c):
            dv = plsc.load_gather(db, [jnp.full((16,), r, jnp.int32)])
            xb[r, pl.ds(0, 16)] = xb[r, pl.ds(0, 16)] * dv
            return c

        lax.fori_loop(0, 1280, rbody, 0)
        pltpu.sync_copy(xb, y1r_hbm.at[pl.ds(r0, 1280)])

        def tch(c, _):
            def trow(r, _2):
                v = xb[c * 128 + r, pl.ds(0, 16)]
                plsc.store_scatter(
                    tb, [iota16, jnp.full((16,), r, jnp.int32)], v)
                return _2

            lax.fori_loop(0, 128, trow, 0)
            pltpu.sync_copy(
                tb, y1rt_hbm.at[pl.ds(0, 16), pl.ds(r0 + c * 128, 128)])
            return _

        lax.fori_loop(0, 10, tch, 0)

        @pl.when(cid == 0)
        def _():
            t0 = sid * 640
            pltpu.sync_copy(xt_hbm.at[pl.ds(t0, 640)], xtb)
            pltpu.sync_copy(dis_hbm.at[pl.ds(4 * NP + t0, 640)], dtb)

            def rbody2(r, c):
                dv = plsc.load_gather(dtb, [jnp.full((16,), r, jnp.int32)])
                for f in range(3):
                    xtb[r, pl.ds(16 * f, 16)] = (
                        xtb[r, pl.ds(16 * f, 16)] * dv)
                return c

            lax.fori_loop(0, 640, rbody2, 0)
            pltpu.sync_copy(xtb, y1t_hbm.at[pl.ds(t0, 640)])

            def tch2(c, _):
                def trow2(r, _2):
                    rr = jnp.full((16,), r, jnp.int32)
                    for f in range(3):
                        v = xtb[c * 128 + r, pl.ds(16 * f, 16)]
                        plsc.store_scatter(ttb, [16 * f + iota16, rr], v)
                    return _2

                lax.fori_loop(0, 128, trow2, 0)
                pltpu.sync_copy(
                    ttb, y1tt_hbm.at[pl.ds(0, 48), pl.ds(t0 + c * 128, 128)])
                return _

            lax.fori_loop(0, 5, tch2, 0)

    def make_prop(rows, width, kc0, kc1, nb):
        zrows = rows // NT

        @functools.partial(
            pl.kernel,
            out_type=jax.ShapeDtypeStruct((NSC, width, rows), jnp.float32),
            mesh=mesh,
            compiler_params=pltpu.CompilerParams(use_tc_tiling_on_sc=False,
                                                 needs_layout_passes=False),
            scratch_types=[
                pltpu.VMEM((kc0, 128), jnp.int32),
                pltpu.VMEM((kc0, 128), jnp.int32),
                pltpu.VMEM((2 * nb, 128, width), jnp.float32),
                pltpu.SemaphoreType.DMA,
                pltpu.SemaphoreType.DMA,
                pltpu.SemaphoreType.DMA,
                pltpu.SemaphoreType.DMA,
                pltpu.VMEM((128, width), jnp.float32),
                pltpu.VMEM((width, 128), jnp.float32),
                pltpu.VMEM_SHARED((rows, width), jnp.float32),
            ],
        )
        def prop(tab_hbm, src_hbm, dst_hbm, zeros_hbm, out_hbm,
                 src_v, dst_v, bufs, sem_g0, sem_g1, sem_s0, sem_s1,
                 slab, tbuf, acc_sh):
            if True:
                cid = lax.axis_index("c")
                sid = lax.axis_index("s")
                base = sid * (kc0 + kc1) + cid * kc0
                kcv = kc0 - cid * (kc0 - kc1)
                r0 = sid * zrows
                pltpu.sync_copy(zeros_hbm, acc_sh.at[pl.ds(r0, zrows)])
                pltpu.sync_copy(src_hbm.at[pl.ds(base, kc0)], src_v)
                pltpu.sync_copy(dst_hbm.at[pl.ds(base, kc0)], dst_v)
                plsc.subcore_barrier()

                def fire_g(j, b, sg):
                    pltpu.async_copy(tab_hbm.at[src_v.at[j]], bufs.at[b], sg)

                def drain_g(j, b, sg):
                    pltpu.make_async_copy(tab_hbm.at[src_v.at[j]],
                                          bufs.at[b], sg).wait()

                def fire_s(j, b, ss):
                    pltpu.async_copy(bufs.at[b], acc_sh.at[dst_v.at[j]],
                                     ss, add=True)

                def drain_s(j, b, ss):
                    pltpu.make_async_copy(bufs.at[b],
                                          acc_sh.at[dst_v.at[j]],
                                          ss).wait()

                for b in range(nb):
                    fire_g(b, b, sem_g0)
                for b in range(nb, 2 * nb):
                    fire_g(b, b, sem_g1)

                def body(jj, c):
                    j0 = jj * (2 * nb)
                    jn = lax.rem(j0 + 2 * nb, kcv)
                    for b in range(nb):
                        drain_g(j0 + b, b, sem_g0)
                    for b in range(nb):
                        fire_s(j0 + b, b, sem_s0)
                    for b in range(nb, 2 * nb):
                        drain_g(j0 + b, b, sem_g1)
                    for b in range(nb, 2 * nb):
                        fire_s(j0 + b, b, sem_s1)
                    for b in range(nb):
                        drain_s(j0 + b, b, sem_s0)
                    for b in range(nb):
                        fire_g(jn + b, b, sem_g0)
                    for b in range(nb, 2 * nb):
                        drain_s(j0 + b, b, sem_s1)
                    for b in range(nb, 2 * nb):
                        fire_g(jn + b, b, sem_g1)
                    return c

                lax.fori_loop(0, kcv // (2 * nb), body, 0)
                for b in range(nb):
                    drain_g(b, b, sem_g0)
                for b in range(nb, 2 * nb):
                    drain_g(b, b, sem_g1)
                plsc.subcore_barrier()
                iota16 = lax.broadcasted_iota(jnp.int32, (16,), 0)

                def fch(c, _):
                    pltpu.sync_copy(acc_sh.at[pl.ds(r0 + c * 128, 128)],
                                    slab)

                    def frow(r, _2):
                        rr = jnp.full((16,), r, jnp.int32)
                        for f in range(width // 16):
                            v = slab[r, pl.ds(16 * f, 16)]
                            plsc.store_scatter(tbuf, [16 * f + iota16, rr],
                                               v)
                        return _2

                    lax.fori_loop(0, 128, frow, 0)
                    pltpu.sync_copy(
                        tbuf,
                        out_hbm.at[cid].at[pl.ds(0, width),
                                           pl.ds(r0 + c * 128, 128)])
                    return _

                lax.fori_loop(0, zrows // 128, fch, 0)

        return prop

    sc_prop_rel = make_prop(ROWS_R, 16, KCR0, KCR1, 8)
    sc_prop_tree = make_prop(NP, 48, KCT0, KCT1, 4)

    @functools.partial(
        pl.kernel,
        out_type=jax.ShapeDtypeStruct((5, NSC, 2, 64, NP), jnp.float32),
        mesh=mesh,
        compiler_params=pltpu.CompilerParams(use_tc_tiling_on_sc=False,
                                             needs_layout_passes=False),
        scratch_types=[
            pltpu.VMEM((NP,), jnp.float32),
            pltpu.VMEM((NP,), jnp.int32),
            pltpu.VMEM((KE, 128), jnp.int32),
            pltpu.VMEM((KE, 128), jnp.int32),
            pltpu.VMEM((KE, 128), jnp.int32),
            pltpu.VMEM((KE, 128), jnp.float32),
            pltpu.VMEM((KE, 128), jnp.int32),
            pltpu.VMEM((KE, 128), jnp.float32),
            pltpu.VMEM_SHARED((ACC_E,), jnp.float32),
            pltpu.SemaphoreType.DMA,
        ],
    )
    def sc_coeff(dis_hbm, bat_hbm, src_hbm, dst_hbm, zeros_hbm, out_hbm,
                 dis_v, bat_v, src_v, dst_v, idx_v, val_v, idx_v2, val_v2,
                 acc_sh, sem):
        if True:
            cid = lax.axis_index("c")
            sid = lax.axis_index("s")
            for g in range(5):
                pltpu.sync_copy(dis_hbm.at[pl.ds(g * NP, NP)], dis_v)
                pltpu.sync_copy(bat_hbm.at[g], bat_v)
                tix = g * NT + sid
                pltpu.sync_copy(src_hbm.at[tix], src_v)
                pltpu.sync_copy(dst_hbm.at[tix], dst_v)
                zd = pltpu.async_copy(
                    zeros_hbm, acc_sh.at[pl.ds(sid * CH_E, CH_E)], sem)
                base = cid * 128

                def cbody(j, c):
                    for k in range(8):
                        sl = pl.ds(k * 16, 16)
                        s16 = src_v[j, sl]
                        d16 = dst_v[j, sl]
                        v = plsc.load_gather(dis_v, [d16])
                        gg = plsc.load_gather(bat_v, [d16])
                        rel = gg - base
                        inh0 = (rel >= 0) & (rel < 64)
                        inh1 = (rel >= 64) & (rel < 128)
                        col0 = jnp.clip(rel, 0, 63)
                        col1 = jnp.clip(rel - 64, 0, 63)
                        idx_v[j, sl] = col0 * NP + s16
                        val_v[j, sl] = jnp.where(inh0, v, jnp.float32(0.0))
                        idx_v2[j, sl] = col1 * NP + s16
                        val_v2[j, sl] = jnp.where(inh1, v, jnp.float32(0.0))
                    return c

                lax.fori_loop(0, KE, cbody, 0)
                for q in range(2):
                    iv = idx_v if q == 0 else idx_v2
                    vv = val_v if q == 0 else val_v2
                    zd.wait()
                    plsc.subcore_barrier()

                    def sbody(j, c):
                        pltpu.async_copy(vv.at[j], acc_sh.at[iv.at[j]],
                                         sem, add=True)
                        return c

                    lax.fori_loop(0, KE, sbody, 0)

                    def dsbody(j, c):
                        pltpu.make_async_copy(vv.at[j], acc_sh.at[iv.at[j]],
                                              sem).wait()
                        return c

                    lax.fori_loop(0, KE, dsbody, 0)
                    plsc.subcore_barrier()
                    fd = []
                    for i in range(4):
                        fd.append(pltpu.async_copy(
                            acc_sh.at[pl.ds(sid * CH_E + i * NP, NP)],
                            out_hbm.at[g].at[cid].at[q].at[sid * 4 + i],
                            sem))
                    for i in range(4):
                        fd[i].wait()
                    if q == 0:
                        zd = pltpu.async_copy(
                            zeros_hbm, acc_sh.at[pl.ds(sid * CH_E, CH_E)],
                            sem)

    return sc_degree, sc_build, sc_prop_rel, sc_prop_tree, sc_coeff


def _k1_body(p0_ref, p1_ref, dis_ref):
    b = pl.program_id(0)
    deg = p0_ref[...] + p1_ref[...] + 1.0
    rowid = lax.rem(b * 18432 + lax.broadcasted_iota(jnp.int32, (18432,), 0),
                    NP)
    dis_ref[...] = jnp.where(rowid < N, lax.rsqrt(deg), 0.0)


def _tc_dis(p0, p1):
    return pl.pallas_call(
        _k1_body,
        grid=(ACC_A // 18432,),
        in_specs=[
            pl.BlockSpec((18432,), lambda i: (i,)),
            pl.BlockSpec((18432,), lambda i: (i,)),
        ],
        out_specs=pl.BlockSpec((18432,), lambda i: (i,)),
        out_shape=jax.ShapeDtypeStruct((ACC_A,), jnp.float32),
    )(p0, p1)


def _k1c_body(c0_ref, c1_ref, out_ref):
    out_ref[...] = 1.0 / jnp.maximum(c0_ref[...] + c1_ref[...], 1.0)


def _tc_invcnt(c0, c1):
    return pl.pallas_call(
        _k1c_body,
        out_shape=jax.ShapeDtypeStruct((5, GP), jnp.float32),
    )(c0, c1)


def _k2_body(z0_ref, z1_ref, y1_ref, dis_ref, w1_ref, b1_ref, y2_ref):
    for i in range(8):
        di = dis_ref[pl.ds(128 * i, 128)]
        zs = (z0_ref[:, i, :] + z1_ref[:, i, :] + y1_ref[:, i, :]) * di[None, :]
        h = lax.dot_general(zs, w1_ref[0], (((0,), (0,)), ((), ())),
                            preferred_element_type=jnp.float32) + b1_ref[0, 0, :]
        y2_ref[pl.ds(128 * i, 128), :] = jnp.maximum(h, 0.0) * di[:, None]


def _tc_layer1(z0, z1, y1t, dis, w1s, b1s, ngraph):
    width = y1t.shape[0]
    rows = y1t.shape[1] * 128
    nblk = (rows // ngraph) // 1024
    return pl.pallas_call(
        _k2_body,
        grid=(rows // 1024,),
        in_specs=[
            pl.BlockSpec((width, 8, 128), lambda i: (0, i, 0)),
            pl.BlockSpec((width, 8, 128), lambda i: (0, i, 0)),
            pl.BlockSpec((width, 8, 128), lambda i: (0, i, 0)),
            pl.BlockSpec((1024,), lambda i: (i,)),
            pl.BlockSpec((1, width, 256), lambda i: (i // nblk, 0, 0)),
            pl.BlockSpec((1, 8, 256), lambda i: (i // nblk, 0, 0)),
        ],
        out_specs=pl.BlockSpec((1024, 256), lambda i: (i, 0)),
        out_shape=jax.ShapeDtypeStruct((rows, 256), jnp.float32),
    )(z0, z1, y1t, dis, w1s, b1s)


def _k3_body(ct_ref, y2_ref, ic_ref, w2_ref, b2_ref, out_ref):
    k = pl.program_id(1)

    @pl.when(k == 0)
    def _():
        out_ref[...] = jnp.zeros_like(out_ref)

    acc = jnp.zeros((256, 256), jnp.float32)
    for i in range(8):
        a = ct_ref[0, :, :, :, i, :].reshape(256, 128)
        b = y2_ref[0, pl.ds(128 * i, 128), :]
        acc += jnp.dot(a, b, preferred_element_type=jnp.float32)
    out_ref[0] += acc

    @pl.when(k == 9)
    def _():
        f = out_ref[0] * ic_ref[0, 0, :][:, None]
        out_ref[0] = jnp.dot(f, w2_ref[0],
                             preferred_element_type=jnp.float32) + b2_ref[0, 0, :]


def _tc_pool_head(ct, y2, ic, w2s, b2s):
    ng = y2.shape[0]
    return pl.pallas_call(
        _k3_body,
        grid=(ng, 10),
        in_specs=[
            pl.BlockSpec((1, 2, 2, 64, 8, 128),
                         lambda g, k: (g, 0, 0, 0, k, 0)),
            pl.BlockSpec((1, 1024, 256), lambda g, k: (g, k, 0)),
            pl.BlockSpec((1, 8, 256), lambda g, k: (g, 0, 0)),
            pl.BlockSpec((1, 256, 256), lambda g, k: (g, 0, 0)),
            pl.BlockSpec((1, 8, 256), lambda g, k: (g, 0, 0)),
        ],
        out_specs=pl.BlockSpec((1, 256, 256), lambda g, k: (g, 0, 0)),
        out_shape=jax.ShapeDtypeStruct((ng, 256, 256), jnp.float32),
    )(ct, y2, ic, w2s, b2s)


def _mlp_body(h_ref, w1_ref, b1_ref, w2_ref, b2_ref, w3_ref, b3_ref, out_ref):
    h = h_ref[...]
    h = jnp.maximum(jnp.dot(h, w1_ref[...], preferred_element_type=jnp.float32)
                    + b1_ref[0, :], 0.0)
    h = jnp.maximum(jnp.dot(h, w2_ref[...], preferred_element_type=jnp.float32)
                    + b2_ref[0, :], 0.0)
    out_ref[...] = jnp.dot(h, w3_ref[...],
                           preferred_element_type=jnp.float32) + b3_ref[0, :]


def _tc_mlp(h, w1, b1, w2, b2, w3, b3):
    return pl.pallas_call(
        _mlp_body,
        out_shape=jax.ShapeDtypeStruct((G, 128), jnp.float32),
    )(h, w1, b1, w2, b2, w3, b3)


def _pad1(a, length, val):
    return jnp.concatenate(
        [a, jnp.full((length - a.shape[0],), val, jnp.int32)])


def _padx(x, width):
    return jnp.pad(x, ((0, NP - x.shape[0]), (0, width - x.shape[1])))


def _bpad(b):
    return jnp.broadcast_to(b, (8, b.shape[0]))


def kernel(relation_left_x, relation_left_edge_index, relation_left_batch,
           W_left_1, b_left_1, W_left_2, b_left_2,
           relation_right_x, relation_right_edge_index, relation_right_batch,
           W_right_1, b_right_1, W_right_2, b_right_2,
           relation_front_x, relation_front_edge_index, relation_front_batch,
           W_front_1, b_front_1, W_front_2, b_front_2,
           relation_behind_x, relation_behind_edge_index, relation_behind_batch,
           W_behind_1, b_behind_1, W_behind_2, b_behind_2,
           question_tree_x, question_tree_edge_index, question_tree_batch,
           W_tree_1, b_tree_1, W_tree_2, b_tree_2,
           W_fc1, b_fc1, W_fc2, b_fc2, W_fc3, b_fc3):
    sc_degree, sc_build, sc_prop_rel, sc_prop_tree, sc_coeff = _sc_kernels()

    xs = [relation_left_x, relation_right_x, relation_front_x,
          relation_behind_x, question_tree_x]
    eis = [relation_left_edge_index, relation_right_edge_index,
           relation_front_edge_index, relation_behind_edge_index,
           question_tree_edge_index]
    batches = [relation_left_batch, relation_right_batch,
               relation_front_batch, relation_behind_batch,
               question_tree_batch]
    w1s = [W_left_1, W_right_1, W_front_1, W_behind_1, W_tree_1]
    b1s = [b_left_1, b_right_1, b_front_1, b_behind_1, b_tree_1]
    w2s = [W_left_2, W_right_2, W_front_2, W_behind_2, W_tree_2]
    b2s = [b_left_2, b_right_2, b_front_2, b_behind_2, b_tree_2]

    eis = [ei.astype(jnp.int32) for ei in eis]
    batches = [b.astype(jnp.int32) for b in batches]

    parts = [g * NP + eis[g][1] for g in range(5)]
    parts += [5 * NP + g * GP + batches[g] for g in range(5)]
    idxa = _pad1(jnp.concatenate(parts), EA, N).reshape(NW, KA, 128)

    acc_a = sc_degree(idxa,
                      jnp.zeros((CH_A,), jnp.float32),
                      jnp.ones((128,), jnp.float32))
    deg_p = acc_a[:, :5 * NP]
    cnt_p = acc_a[:, 5 * NP:5 * NP + 5 * GP].reshape(NSC, 5, GP)
    inv_cnt = _tc_invcnt(cnt_p[0], cnt_p[1])
    ic = jnp.broadcast_to(inv_cnt[:, None, :G], (5, 8, G))

    dis_all = _tc_dis(acc_a[0], acc_a[1])
    xr = jnp.concatenate([_padx(xs[g], 16) for g in range(4)], axis=0)
    xt = _padx(xs[4], 48)
    y1r, y1rt, y1t, y1tt = sc_build(xr, xt, dis_all)

    srcc = _pad1(jnp.concatenate([g * NP + eis[g][0] for g in range(4)]),
                 SR_R * 128, N).reshape(SR_R, 128)
    dstc = _pad1(jnp.concatenate([g * NP + eis[g][1] for g in range(4)]),
                 SR_R * 128, N).reshape(SR_R, 128)

    srct = _pad1(eis[4][0], SR_T * 128, N).reshape(SR_T, 128)
    dstt = _pad1(eis[4][1], SR_T * 128, N).reshape(SR_T, 128)
    zt = sc_prop_tree(y1t, srct, dstt, jnp.zeros((ZR_T, 48), jnp.float32))

    w1r = jnp.stack([jnp.pad(w1s[g], ((0, 1), (0, 0))) for g in range(4)])
    b1r = jnp.stack([_bpad(b1s[g]) for g in range(4)])
    w1t = jnp.pad(w1s[4], ((0, 9), (0, 0)))[None]
    b1t = _bpad(b1s[4])[None]
    y2t = _tc_layer1(zt[0].reshape(48, 80, 128), zt[1].reshape(48, 80, 128),
                     y1tt.reshape(48, 80, 128), dis_all[4 * NP:5 * NP],
                     w1t, b1t, 1)

    dis5 = dis_all
    bat5 = jnp.stack([_pad1(b, NP, 0) for b in batches])
    loop = jnp.arange(N, dtype=jnp.int32)
    srce = jnp.stack([_pad1(jnp.concatenate([eis[g][0], loop]),
                            NT * KE * 128, N) for g in range(5)])
    dste = jnp.stack([_pad1(jnp.concatenate([eis[g][1], loop]),
                            NT * KE * 128, N) for g in range(5)])
    srce = srce.reshape(5 * NT, KE, 128)
    dste = dste.reshape(5 * NT, KE, 128)

    ze = jnp.zeros((CH_E,), jnp.float32) + y2t[0, 0] * 0.0
    co = sc_coeff(dis5, bat5, srce, dste, ze)
    ct = co.reshape(5, 2, 2, 64, 80, 128)

    zc = jnp.zeros((ZR_R, 16), jnp.float32) + co[0, 0, 0, 0, 0] * 0.0
    zr = sc_prop_rel(y1r, srcc, dstc, zc)
    y2r = _tc_layer1(zr[0].reshape(16, 320, 128), zr[1].reshape(16, 320, 128),
                     y1rt.reshape(16, 320, 128), dis_all[:4 * NP],
                     w1r, b1r, 4)

    w2r = jnp.stack(w2s[:4])
    b2r = jnp.stack([_bpad(b) for b in b2s[:4]])
    featr = _tc_pool_head(ct[:4], y2r.reshape(4, NP, 256), ic[:4], w2r, b2r)
    featt = _tc_pool_head(ct[4:], y2t.reshape(1, NP, 256), ic[4:],
                          w2s[4][None], _bpad(b2s[4])[None])

    h = jnp.concatenate([featr[0], featr[1], featr[2], featr[3], featt[0]],
                        axis=1)
    w3p = jnp.pad(W_fc3, ((0, 0), (0, 128 - 32)))
    b3p = _bpad(jnp.pad(b_fc3, (0, 128 - 32)))
    out = _tc_mlp(h, W_fc1, _bpad(b_fc1), W_fc2, _bpad(b_fc2), w3p, b3p)
    return out[:, :32]


# speedup vs baseline: 1.7055x; 1.7055x over previous
import functools

import jax
import jax.numpy as jnp
from jax import lax
from jax.experimental import pallas as pl
from jax.experimental.pallas import tpu as pltpu
from jax.experimental.pallas import tpu_sc as plsc

N = 10000
E = 160000
G = 256
NP = 10240
GP = 512
NSC = 2
NT = 16
NW = NSC * NT

ACC_A = 55296
CH_A = ACC_A // NT
KA = 208
EA = NW * KA * 128

KCR0, KCR1 = 160, 160
KCT0, KCT1 = 40, 40
SR_R = 16 * (KCR0 + KCR1) + KCR0
SR_T = 16 * (KCT0 + KCT1) + KCT0
ROWS_R = 4 * NP
ZR_R = ROWS_R // NT
ZR_T = NP // NT

KE = 84
ACC_E = 64 * NP
CH_E = ACC_E // NT


@functools.cache
def _sc_kernels():
    mesh = plsc.VectorSubcoreMesh(core_axis_name="c", subcore_axis_name="s",
                                  num_cores=NSC, num_subcores=NT)

    @functools.partial(
        pl.kernel,
        out_type=jax.ShapeDtypeStruct((NSC, ACC_A), jnp.float32),
        mesh=mesh,
        compiler_params=pltpu.CompilerParams(use_tc_tiling_on_sc=False),
        scratch_types=[
            pltpu.VMEM((KA, 128), jnp.int32),
            pltpu.VMEM((128,), jnp.float32),
            pltpu.VMEM_SHARED((ACC_A,), jnp.float32),
            pltpu.SemaphoreType.DMA,
        ],
    )
    def sc_degree(idx_hbm, zeros_hbm, ones_hbm, out_hbm, idx_v, ones_v,
                  acc_sh, sem):
        if True:
            cid = lax.axis_index("c")
            sid = lax.axis_index("s")
            wid = sid * NSC + cid
            pltpu.sync_copy(zeros_hbm, acc_sh.at[pl.ds(sid * CH_A, CH_A)])
            pltpu.sync_copy(ones_hbm, ones_v)
            pltpu.sync_copy(idx_hbm.at[wid], idx_v)
            plsc.subcore_barrier()

            def body(j, c):
                pltpu.async_copy(ones_v, acc_sh.at[idx_v.at[j]], sem,
                                 add=True)
                return c

            lax.fori_loop(0, KA, body, 0)

            def dbody(j, c):
                pltpu.make_async_copy(ones_v, acc_sh.at[idx_v.at[j]],
                                      sem).wait()
                return c

            lax.fori_loop(0, KA, dbody, 0)
            plsc.subcore_barrier()
            pltpu.sync_copy(acc_sh.at[pl.ds(sid * CH_A, CH_A)],
                            out_hbm.at[cid].at[pl.ds(sid * CH_A, CH_A)])

    @functools.partial(
        pl.kernel,
        out_type=(
            jax.ShapeDtypeStruct((4 * NP, 16), jnp.float32),
            jax.ShapeDtypeStruct((16, 4 * NP), jnp.float32),
            jax.ShapeDtypeStruct((NP, 48), jnp.float32),
            jax.ShapeDtypeStruct((48, NP), jnp.float32),
        ),
        mesh=mesh,
        compiler_params=pltpu.CompilerParams(use_tc_tiling_on_sc=False,
                                             needs_layout_passes=False),
        scratch_types=[
            pltpu.VMEM((1280, 16), jnp.float32),
            pltpu.VMEM((1280,), jnp.float32),
            pltpu.VMEM((16, 128), jnp.float32),
            pltpu.VMEM((640, 48), jnp.float32),
            pltpu.VMEM((640,), jnp.float32),
            pltpu.VMEM((48, 128), jnp.float32),
        ],
    )
    def sc_build(xr_hbm, xt_hbm, dis_hbm, y1r_hbm, y1rt_hbm, y1t_hbm,
                 y1tt_hbm, xb, db, tb, xtb, dtb, ttb):
        cid = lax.axis_index("c")
        sid = lax.axis_index("s")
        wid = sid * NSC + cid
        iota16 = lax.broadcasted_iota(jnp.int32, (16,), 0)
        r0 = wid * 1280
        pltpu.sync_copy(xr_hbm.at[pl.ds(r0, 1280)], xb)
        pltpu.sync_copy(dis_hbm.at[pl.ds(r0, 1280)], db)

        def rbody(r, c):
            dv = plsc.load_gather(db, [jnp.full((16,), r, jnp.int32)])
            xb[r, pl.ds(0, 16)] = xb[r, pl.ds(0, 16)] * dv
            return c

        lax.fori_loop(0, 1280, rbody, 0)
        pltpu.sync_copy(xb, y1r_hbm.at[pl.ds(r0, 1280)])

        def tch(c, _):
            def trow(r, _2):
                v = xb[c * 128 + r, pl.ds(0, 16)]
                plsc.store_scatter(
                    tb, [iota16, jnp.full((16,), r, jnp.int32)], v)
                return _2

            lax.fori_loop(0, 128, trow, 0)
            pltpu.sync_copy(
                tb, y1rt_hbm.at[pl.ds(0, 16), pl.ds(r0 + c * 128, 128)])
            return _

        lax.fori_loop(0, 10, tch, 0)

        @pl.when(cid == 0)
        def _():
            t0 = sid * 640
            pltpu.sync_copy(xt_hbm.at[pl.ds(t0, 640)], xtb)
            pltpu.sync_copy(dis_hbm.at[pl.ds(4 * NP + t0, 640)], dtb)

            def rbody2(r, c):
                dv = plsc.load_gather(dtb, [jnp.full((16,), r, jnp.int32)])
                for f in range(3):
                    xtb[r, pl.ds(16 * f, 16)] = (
                        xtb[r, pl.ds(16 * f, 16)] * dv)
                return c

            lax.fori_loop(0, 640, rbody2, 0)
            pltpu.sync_copy(xtb, y1t_hbm.at[pl.ds(t0, 640)])

            def tch2(c, _):
                def trow2(r, _2):
                    rr = jnp.full((16,), r, jnp.int32)
                    for f in range(3):
                        v = xtb[c * 128 + r, pl.ds(16 * f, 16)]
                        plsc.store_scatter(ttb, [16 * f + iota16, rr], v)
                    return _2

                lax.fori_loop(0, 128, trow2, 0)
                pltpu.sync_copy(
                    ttb, y1tt_hbm.at[pl.ds(0, 48), pl.ds(t0 + c * 128, 128)])
                return _

            lax.fori_loop(0, 5, tch2, 0)

    def make_prop(rows, width, kc0, kc1, nb):
        zrows = rows // NT

        @functools.partial(
            pl.kernel,
            out_type=jax.ShapeDtypeStruct((NSC, width, rows), jnp.float32),
            mesh=mesh,
            compiler_params=pltpu.CompilerParams(use_tc_tiling_on_sc=False,
                                                 needs_layout_passes=False),
            scratch_types=[
                pltpu.VMEM((kc0, 128), jnp.int32),
                pltpu.VMEM((kc0, 128), jnp.int32),
                pltpu.VMEM((2 * nb, 128, width), jnp.float32),
                pltpu.SemaphoreType.DMA,
                pltpu.SemaphoreType.DMA,
                pltpu.SemaphoreType.DMA,
                pltpu.SemaphoreType.DMA,
                pltpu.VMEM((128, width), jnp.float32),
                pltpu.VMEM((width, 128), jnp.float32),
                pltpu.VMEM_SHARED((rows, width), jnp.float32),
            ],
        )
        def prop(tab_hbm, src_hbm, dst_hbm, zeros_hbm, out_hbm,
                 src_v, dst_v, bufs, sem_g0, sem_g1, sem_s0, sem_s1,
                 slab, tbuf, acc_sh):
            if True:
                cid = lax.axis_index("c")
                sid = lax.axis_index("s")
                if kc0 == kc1:
                    base = (sid * NSC + cid) * kc0
                    kcv = kc0
                else:
                    base = sid * (kc0 + kc1) + cid * kc0
                    kcv = kc0 - cid * (kc0 - kc1)
                r0 = sid * zrows
                pltpu.sync_copy(zeros_hbm, acc_sh.at[pl.ds(r0, zrows)])
                pltpu.sync_copy(src_hbm.at[pl.ds(base, kc0)], src_v)
                pltpu.sync_copy(dst_hbm.at[pl.ds(base, kc0)], dst_v)
                plsc.subcore_barrier()

                def fire_g(j, b, sg):
                    pltpu.async_copy(tab_hbm.at[src_v.at[j]], bufs.at[b], sg)

                def drain_g(j, b, sg):
                    pltpu.make_async_copy(tab_hbm.at[src_v.at[j]],
                                          bufs.at[b], sg).wait()

                def fire_s(j, b, ss):
                    pltpu.async_copy(bufs.at[b], acc_sh.at[dst_v.at[j]],
                                     ss, add=True)

                def drain_s(j, b, ss):
                    pltpu.make_async_copy(bufs.at[b],
                                          acc_sh.at[dst_v.at[j]],
                                          ss).wait()

                for b in range(nb):
                    fire_g(b, b, sem_g0)
                for b in range(nb, 2 * nb):
                    fire_g(b, b, sem_g1)

                def body(jj, c):
                    j0 = jj * (2 * nb)
                    jn = lax.rem(j0 + 2 * nb, kcv)
                    for b in range(nb):
                        drain_g(j0 + b, b, sem_g0)
                    for b in range(nb):
                        fire_s(j0 + b, b, sem_s0)
                    for b in range(nb, 2 * nb):
                        drain_g(j0 + b, b, sem_g1)
                    for b in range(nb, 2 * nb):
                        fire_s(j0 + b, b, sem_s1)
                    for b in range(nb):
                        drain_s(j0 + b, b, sem_s0)
                    for b in range(nb):
                        fire_g(jn + b, b, sem_g0)
                    for b in range(nb, 2 * nb):
                        drain_s(j0 + b, b, sem_s1)
                    for b in range(nb, 2 * nb):
                        fire_g(jn + b, b, sem_g1)
                    return c

                lax.fori_loop(0, kcv // (2 * nb), body, 0)
                for b in range(nb):
                    drain_g(b, b, sem_g0)
                for b in range(nb, 2 * nb):
                    drain_g(b, b, sem_g1)
                plsc.subcore_barrier()
                iota16 = lax.broadcasted_iota(jnp.int32, (16,), 0)

                def fch(c, _):
                    pltpu.sync_copy(acc_sh.at[pl.ds(r0 + c * 128, 128)],
                                    slab)

                    def frow(r, _2):
                        rr = jnp.full((16,), r, jnp.int32)
                        for f in range(width // 16):
                            v = slab[r, pl.ds(16 * f, 16)]
                            plsc.store_scatter(tbuf, [16 * f + iota16, rr],
                                               v)
                        return _2

                    lax.fori_loop(0, 128, frow, 0)
                    pltpu.sync_copy(
                        tbuf,
                        out_hbm.at[cid].at[pl.ds(0, width),
                                           pl.ds(r0 + c * 128, 128)])
                    return _

                lax.fori_loop(0, zrows // 128, fch, 0)

        return prop

    sc_prop_rel = make_prop(ROWS_R, 16, KCR0, KCR1, 8)
    sc_prop_tree = make_prop(NP, 48, KCT0, KCT1, 4)

    @functools.partial(
        pl.kernel,
        out_type=jax.ShapeDtypeStruct((5, NSC, 2, 64, NP), jnp.float32),
        mesh=mesh,
        compiler_params=pltpu.CompilerParams(use_tc_tiling_on_sc=False,
                                             needs_layout_passes=False),
        scratch_types=[
            pltpu.VMEM((NP,), jnp.float32),
            pltpu.VMEM((NP,), jnp.int32),
            pltpu.VMEM((KE, 128), jnp.int32),
            pltpu.VMEM((KE, 128), jnp.int32),
            pltpu.VMEM((KE, 128), jnp.int32),
            pltpu.VMEM((KE, 128), jnp.float32),
            pltpu.VMEM((KE, 128), jnp.int32),
            pltpu.VMEM((KE, 128), jnp.float32),
            pltpu.VMEM_SHARED((ACC_E,), jnp.float32),
            pltpu.SemaphoreType.DMA,
        ],
    )
    def sc_coeff(dis_hbm, bat_hbm, src_hbm, dst_hbm, zeros_hbm, out_hbm,
                 dis_v, bat_v, src_v, dst_v, idx_v, val_v, idx_v2, val_v2,
                 acc_sh, sem):
        if True:
            cid = lax.axis_index("c")
            sid = lax.axis_index("s")
            for g in range(5):
                pltpu.sync_copy(dis_hbm.at[pl.ds(g * NP, NP)], dis_v)
                pltpu.sync_copy(bat_hbm.at[g], bat_v)
                tix = g * NT + sid
                pltpu.sync_copy(src_hbm.at[tix], src_v)
                pltpu.sync_copy(dst_hbm.at[tix], dst_v)
                zd = pltpu.async_copy(
                    zeros_hbm, acc_sh.at[pl.ds(sid * CH_E, CH_E)], sem)
                base = cid * 128

                def cbody(j, c):
                    for k in range(8):
                        sl = pl.ds(k * 16, 16)
                        s16 = src_v[j, sl]
                        d16 = dst_v[j, sl]
                        v = plsc.load_gather(dis_v, [d16])
                        gg = plsc.load_gather(bat_v, [d16])
                        rel = gg - base
                        inh0 = (rel >= 0) & (rel < 64)
                        inh1 = (rel >= 64) & (rel < 128)
                        col0 = jnp.clip(rel, 0, 63)
                        col1 = jnp.clip(rel - 64, 0, 63)
                        idx_v[j, sl] = col0 * NP + s16
                        val_v[j, sl] = jnp.where(inh0, v, jnp.float32(0.0))
                        idx_v2[j, sl] = col1 * NP + s16
                        val_v2[j, sl] = jnp.where(inh1, v, jnp.float32(0.0))
                    return c

                lax.fori_loop(0, KE, cbody, 0)
                for q in range(2):
                    iv = idx_v if q == 0 else idx_v2
                    vv = val_v if q == 0 else val_v2
                    zd.wait()
                    plsc.subcore_barrier()

                    def sbody(j, c):
                        pltpu.async_copy(vv.at[j], acc_sh.at[iv.at[j]],
                                         sem, add=True)
                        return c

                    lax.fori_loop(0, KE, sbody, 0)

                    def dsbody(j, c):
                        pltpu.make_async_copy(vv.at[j], acc_sh.at[iv.at[j]],
                                              sem).wait()
                        return c

                    lax.fori_loop(0, KE, dsbody, 0)
                    plsc.subcore_barrier()
                    fd = []
                    for i in range(4):
                        fd.append(pltpu.async_copy(
                            acc_sh.at[pl.ds(sid * CH_E + i * NP, NP)],
                            out_hbm.at[g].at[cid].at[q].at[sid * 4 + i],
                            sem))
                    for i in range(4):
                        fd[i].wait()
                    if q == 0:
                        zd = pltpu.async_copy(
                            zeros_hbm, acc_sh.at[pl.ds(sid * CH_E, CH_E)],
                            sem)

    return sc_degree, sc_build, sc_prop_rel, sc_prop_tree, sc_coeff


def _k1_body(p0_ref, p1_ref, dis_ref):
    b = pl.program_id(0)
    deg = p0_ref[...] + p1_ref[...] + 1.0
    rowid = lax.rem(b * 18432 + lax.broadcasted_iota(jnp.int32, (18432,), 0),
                    NP)
    dis_ref[...] = jnp.where(rowid < N, lax.rsqrt(deg), 0.0)


def _tc_dis(p0, p1):
    return pl.pallas_call(
        _k1_body,
        grid=(ACC_A // 18432,),
        in_specs=[
            pl.BlockSpec((18432,), lambda i: (i,)),
            pl.BlockSpec((18432,), lambda i: (i,)),
        ],
        out_specs=pl.BlockSpec((18432,), lambda i: (i,)),
        out_shape=jax.ShapeDtypeStruct((ACC_A,), jnp.float32),
    )(p0, p1)


def _k1c_body(c0_ref, c1_ref, out_ref):
    out_ref[...] = 1.0 / jnp.maximum(c0_ref[...] + c1_ref[...], 1.0)


def _tc_invcnt(c0, c1):
    return pl.pallas_call(
        _k1c_body,
        out_shape=jax.ShapeDtypeStruct((5, GP), jnp.float32),
    )(c0, c1)


def _k2_body(z0_ref, z1_ref, y1_ref, dis_ref, w1_ref, b1_ref, y2_ref):
    for i in range(8):
        di = dis_ref[pl.ds(128 * i, 128)]
        zs = (z0_ref[:, i, :] + z1_ref[:, i, :] + y1_ref[:, i, :]) * di[None, :]
        h = lax.dot_general(zs, w1_ref[0], (((0,), (0,)), ((), ())),
                            preferred_element_type=jnp.float32) + b1_ref[0, 0, :]
        y2_ref[pl.ds(128 * i, 128), :] = jnp.maximum(h, 0.0) * di[:, None]


def _tc_layer1(z0, z1, y1t, dis, w1s, b1s, ngraph):
    width = y1t.shape[0]
    rows = y1t.shape[1] * 128
    nblk = (rows // ngraph) // 1024
    return pl.pallas_call(
        _k2_body,
        grid=(rows // 1024,),
        in_specs=[
            pl.BlockSpec((width, 8, 128), lambda i: (0, i, 0)),
            pl.BlockSpec((width, 8, 128), lambda i: (0, i, 0)),
            pl.BlockSpec((width, 8, 128), lambda i: (0, i, 0)),
            pl.BlockSpec((1024,), lambda i: (i,)),
            pl.BlockSpec((1, width, 256), lambda i: (i // nblk, 0, 0)),
            pl.BlockSpec((1, 8, 256), lambda i: (i // nblk, 0, 0)),
        ],
        out_specs=pl.BlockSpec((1024, 256), lambda i: (i, 0)),
        out_shape=jax.ShapeDtypeStruct((rows, 256), jnp.float32),
    )(z0, z1, y1t, dis, w1s, b1s)


def _k3_body(ct_ref, y2_ref, ic_ref, w2_ref, b2_ref, out_ref):
    k = pl.program_id(1)

    @pl.when(k == 0)
    def _():
        out_ref[...] = jnp.zeros_like(out_ref)

    acc = jnp.zeros((256, 256), jnp.float32)
    for i in range(8):
        a = ct_ref[0, :, :, :, i, :].reshape(256, 128)
        b = y2_ref[0, pl.ds(128 * i, 128), :]
        acc += jnp.dot(a, b, preferred_element_type=jnp.float32)
    out_ref[0] += acc

    @pl.when(k == 9)
    def _():
        f = out_ref[0] * ic_ref[0, 0, :][:, None]
        out_ref[0] = jnp.dot(f, w2_ref[0],
                             preferred_element_type=jnp.float32) + b2_ref[0, 0, :]


def _tc_pool_head(ct, y2, ic, w2s, b2s):
    ng = y2.shape[0]
    return pl.pallas_call(
        _k3_body,
        grid=(ng, 10),
        in_specs=[
            pl.BlockSpec((1, 2, 2, 64, 8, 128),
                         lambda g, k: (g, 0, 0, 0, k, 0)),
            pl.BlockSpec((1, 1024, 256), lambda g, k: (g, k, 0)),
            pl.BlockSpec((1, 8, 256), lambda g, k: (g, 0, 0)),
            pl.BlockSpec((1, 256, 256), lambda g, k: (g, 0, 0)),
            pl.BlockSpec((1, 8, 256), lambda g, k: (g, 0, 0)),
        ],
        out_specs=pl.BlockSpec((1, 256, 256), lambda g, k: (g, 0, 0)),
        out_shape=jax.ShapeDtypeStruct((ng, 256, 256), jnp.float32),
    )(ct, y2, ic, w2s, b2s)


def _mlp_body(h_ref, w1_ref, b1_ref, w2_ref, b2_ref, w3_ref, b3_ref, out_ref):
    h = h_ref[...]
    h = jnp.maximum(jnp.dot(h, w1_ref[...], preferred_element_type=jnp.float32)
                    + b1_ref[0, :], 0.0)
    h = jnp.maximum(jnp.dot(h, w2_ref[...], preferred_element_type=jnp.float32)
                    + b2_ref[0, :], 0.0)
    out_ref[...] = jnp.dot(h, w3_ref[...],
                           preferred_element_type=jnp.float32) + b3_ref[0, :]


def _tc_mlp(h, w1, b1, w2, b2, w3, b3):
    return pl.pallas_call(
        _mlp_body,
        out_shape=jax.ShapeDtypeStruct((G, 128), jnp.float32),
    )(h, w1, b1, w2, b2, w3, b3)


def _pad1(a, length, val):
    return jnp.concatenate(
        [a, jnp.full((length - a.shape[0],), val, jnp.int32)])


def _padx(x, width):
    return jnp.pad(x, ((0, NP - x.shape[0]), (0, width - x.shape[1])))


def _bpad(b):
    return jnp.broadcast_to(b, (8, b.shape[0]))


def kernel(relation_left_x, relation_left_edge_index, relation_left_batch,
           W_left_1, b_left_1, W_left_2, b_left_2,
           relation_right_x, relation_right_edge_index, relation_right_batch,
           W_right_1, b_right_1, W_right_2, b_right_2,
           relation_front_x, relation_front_edge_index, relation_front_batch,
           W_front_1, b_front_1, W_front_2, b_front_2,
           relation_behind_x, relation_behind_edge_index, relation_behind_batch,
           W_behind_1, b_behind_1, W_behind_2, b_behind_2,
           question_tree_x, question_tree_edge_index, question_tree_batch,
           W_tree_1, b_tree_1, W_tree_2, b_tree_2,
           W_fc1, b_fc1, W_fc2, b_fc2, W_fc3, b_fc3):
    sc_degree, sc_build, sc_prop_rel, sc_prop_tree, sc_coeff = _sc_kernels()

    xs = [relation_left_x, relation_right_x, relation_front_x,
          relation_behind_x, question_tree_x]
    eis = [relation_left_edge_index, relation_right_edge_index,
           relation_front_edge_index, relation_behind_edge_index,
           question_tree_edge_index]
    batches = [relation_left_batch, relation_right_batch,
               relation_front_batch, relation_behind_batch,
               question_tree_batch]
    w1s = [W_left_1, W_right_1, W_front_1, W_behind_1, W_tree_1]
    b1s = [b_left_1, b_right_1, b_front_1, b_behind_1, b_tree_1]
    w2s = [W_left_2, W_right_2, W_front_2, W_behind_2, W_tree_2]
    b2s = [b_left_2, b_right_2, b_front_2, b_behind_2, b_tree_2]

    eis = [ei.astype(jnp.int32) for ei in eis]
    batches = [b.astype(jnp.int32) for b in batches]

    parts = [g * NP + eis[g][1] for g in range(5)]
    parts += [5 * NP + g * GP + batches[g] for g in range(5)]
    idxa = _pad1(jnp.concatenate(parts), EA, N).reshape(NW, KA, 128)

    acc_a = sc_degree(idxa,
                      jnp.zeros((CH_A,), jnp.float32),
                      jnp.ones((128,), jnp.float32))
    deg_p = acc_a[:, :5 * NP]
    cnt_p = acc_a[:, 5 * NP:5 * NP + 5 * GP].reshape(NSC, 5, GP)
    inv_cnt = _tc_invcnt(cnt_p[0], cnt_p[1])
    ic = jnp.broadcast_to(inv_cnt[:, None, :G], (5, 8, G))

    dis_all = _tc_dis(acc_a[0], acc_a[1])
    xr = jnp.concatenate([_padx(xs[g], 16) for g in range(4)], axis=0)
    xt = _padx(xs[4], 48)
    y1r, y1rt, y1t, y1tt = sc_build(xr, xt, dis_all)

    srcc = _pad1(jnp.concatenate([g * NP + eis[g][0] for g in range(4)]),
                 SR_R * 128, N).reshape(SR_R, 128)
    dstc = _pad1(jnp.concatenate([g * NP + eis[g][1] for g in range(4)]),
                 SR_R * 128, N).reshape(SR_R, 128)

    srct = _pad1(eis[4][0], SR_T * 128, N).reshape(SR_T, 128)
    dstt = _pad1(eis[4][1], SR_T * 128, N).reshape(SR_T, 128)
    zt = sc_prop_tree(y1t, srct, dstt, jnp.zeros((ZR_T, 48), jnp.float32))

    w1r = jnp.stack([jnp.pad(w1s[g], ((0, 1), (0, 0))) for g in range(4)])
    b1r = jnp.stack([_bpad(b1s[g]) for g in range(4)])
    w1t = jnp.pad(w1s[4], ((0, 9), (0, 0)))[None]
    b1t = _bpad(b1s[4])[None]
    y2t = _tc_layer1(zt[0].reshape(48, 80, 128), zt[1].reshape(48, 80, 128),
                     y1tt.reshape(48, 80, 128), dis_all[4 * NP:5 * NP],
                     w1t, b1t, 1)

    dis5 = dis_all
    bat5 = jnp.stack([_pad1(b, NP, 0) for b in batches])
    loop = jnp.arange(N, dtype=jnp.int32)
    srce = jnp.stack([_pad1(jnp.concatenate([eis[g][0], loop]),
                            NT * KE * 128, N) for g in range(5)])
    dste = jnp.stack([_pad1(jnp.concatenate([eis[g][1], loop]),
                            NT * KE * 128, N) for g in range(5)])
    srce = srce.reshape(5 * NT, KE, 128)
    dste = dste.reshape(5 * NT, KE, 128)

    ze = jnp.zeros((CH_E,), jnp.float32) + y2t[0, 0] * 0.0
    co = sc_coeff(dis5, bat5, srce, dste, ze)
    ct = co.reshape(5, 2, 2, 64, 80, 128)

    zc = jnp.zeros((ZR_R, 16), jnp.float32) + co[0, 0, 0, 0, 0] * 0.0
    zr = sc_prop_rel(y1r, srcc, dstc, zc)
    y2r = _tc_layer1(zr[0].reshape(16, 320, 128), zr[1].reshape(16, 320, 128),
                     y1rt.reshape(16, 320, 128), dis_all[:4 * NP],
                     w1r, b1r, 4)

    w2r = jnp.stack(w2s[:4])
    b2r = jnp.stack([_bpad(b) for b in b2s[:4]])
    featr = _tc_pool_head(ct[:4], y2r.reshape(4, NP, 256), ic[:4], w2r, b2r)
    featt = _tc_pool_head(ct[4:], y2t.reshape(1, NP, 256), ic[4:],
                          w2s[4][None], _bpad(b2s[4])[None])

    h = jnp.concatenate([featr[0], featr[1], featr[2], featr[3], featt[0]],
                        axis=1)
    w3p = jnp.pad(W_fc3, ((0, 0), (0, 128 - 32)))
    b3p = _bpad(jnp.pad(b_fc3, (0, 128 - 32)))
    out = _tc_mlp(h, W_fc1, _bpad(b_fc1), W_fc2, _bpad(b_fc2), w3p, b3p)
    return out[:, :32]
